# baseline (speedup 1.0000x reference)
import functools

import jax
import jax.numpy as jnp
from jax import lax
from jax.experimental import pallas as pl
from jax.experimental.pallas import tpu as pltpu
from jax.experimental.pallas import tpu_sc as plsc

K_OUT = 8
LIST_CAP = 10
LANES = 16
S_LEN = 4096
N_GROUPS = S_LEN // LANES
N_CHUNKS = 4
ROWS_PER_CHUNK = S_LEN // N_CHUNKS
GROUPS_PER_CHUNK = N_GROUPS // N_CHUNKS


def _insert(ms, x):
    new = []
    for m in ms:
        hi = jnp.maximum(m, x)
        x = jnp.minimum(m, x)
        new.append(hi)
    return new


def _sc_body(in_hbm, out_hbm, xbuf, gmbuf, glist, obuf, sem):
    info = plsc.get_sparse_core_info()
    nc, ns = info.num_cores, info.num_subcores
    wid = lax.axis_index("s") * nc + lax.axis_index("c")
    tasks_per_tile = (4 * 64) // (nc * ns)
    lane_iota = lax.broadcasted_iota(jnp.int32, (LANES,), 0)
    neg_inf = jnp.full((LANES,), -jnp.inf, dtype=jnp.float32)

    def task_ids(t):
        task = wid * tasks_per_tile + t
        return task, task // 64, task % 64

    def start_task_dmas(t):
        _, b, cg = task_ids(t)
        handles = []
        for q in range(N_CHUNKS):
            handles.append(
                pltpu.async_copy(
                    in_hbm.at[
                        b,
                        pl.ds(q * ROWS_PER_CHUNK, ROWS_PER_CHUNK),
                        pl.ds(cg * LANES, LANES),
                    ],
                    xbuf.at[pl.ds(q * ROWS_PER_CHUNK, ROWS_PER_CHUNK), :],
                    sem[q],
                )
            )
        return handles

    zero16 = jnp.zeros((LANES,), jnp.int32)
    for r in range(LIST_CAP):
        glist[r] = zero16

    handles = start_task_dmas(0)

    for t in range(tasks_per_tile):
        task, _, _ = task_ids(t)

        def gmax(g):
            base = g * LANES
            m01 = jnp.maximum(xbuf[base + 0], xbuf[base + 1])
            m23 = jnp.maximum(xbuf[base + 2], xbuf[base + 3])
            m45 = jnp.maximum(xbuf[base + 4], xbuf[base + 5])
            m67 = jnp.maximum(xbuf[base + 6], xbuf[base + 7])
            m89 = jnp.maximum(xbuf[base + 8], xbuf[base + 9])
            mab = jnp.maximum(xbuf[base + 10], xbuf[base + 11])
            mcd = jnp.maximum(xbuf[base + 12], xbuf[base + 13])
            mef = jnp.maximum(xbuf[base + 14], xbuf[base + 15])
            m = jnp.maximum(
                jnp.maximum(jnp.maximum(m01, m23), jnp.maximum(m45, m67)),
                jnp.maximum(jnp.maximum(m89, mab), jnp.maximum(mcd, mef)),
            )
            gmbuf[g] = m
            return m

        msab = ((neg_inf,) * K_OUT,) * 2
        for q in range(N_CHUNKS):
            handles[q].wait()

            def pair_step(i, carry):
                msa, msb = carry
                g = i * 2
                msa = tuple(_insert(list(msa), gmax(g)))
                msb = tuple(_insert(list(msb), gmax(g + 1)))
                return (msa, msb)

            msab = lax.fori_loop(
                q * (GROUPS_PER_CHUNK // 2),
                (q + 1) * (GROUPS_PER_CHUNK // 2),
                pair_step,
                msab,
            )
        ms = list(msab[0])
        for x in msab[1]:
            ms = _insert(ms, x)
        t8 = ms[K_OUT - 1]

        def list_step(g, cnt):
            sel = jnp.logical_and(gmbuf[g] >= t8, cnt < LIST_CAP)
            gvec = jnp.full((LANES,), g, dtype=jnp.int32)
            plsc.store_scatter(glist, [cnt, lane_iota], gvec, mask=sel)
            return cnt + jnp.where(sel, 1, 0)

        cntf = lax.fori_loop(
            0, N_GROUPS, list_step, jnp.zeros((LANES,), jnp.int32), unroll=2
        )

        def extract_step(j, carry):
            msa, msb = carry
            rows = glist[j] * LANES
            valid = j < cntf
            msa, msb = list(msa), list(msb)
            for v in range(0, LANES, 2):
                xa = plsc.load_gather(xbuf, [rows + v, lane_iota])
                xb = plsc.load_gather(xbuf, [rows + v + 1, lane_iota])
                msa = _insert(msa, jnp.where(valid, xa, neg_inf))
                msb = _insert(msb, jnp.where(valid, xb, neg_inf))
            return (tuple(msa), tuple(msb))

        msa, msb = lax.fori_loop(
            0, LIST_CAP, extract_step, ((neg_inf,) * K_OUT,) * 2
        )
        ms2 = list(msa)
        for x in msb:
            ms2 = _insert(ms2, x)

        for k in range(K_OUT):
            obuf[k] = ms2[k]
        pltpu.sync_copy(obuf, out_hbm.at[task])

        if t + 1 < tasks_per_tile:
            handles = start_task_dmas(t + 1)


def _sc_call(inputs):
    mesh = plsc.VectorSubcoreMesh(core_axis_name="c", subcore_axis_name="s")
    return pl.kernel(
        _sc_body,
        out_type=jax.ShapeDtypeStruct((4 * 64, K_OUT, LANES), jnp.float32),
        mesh=mesh,
        compiler_params=pltpu.CompilerParams(
            use_tc_tiling_on_sc=False, needs_layout_passes=False
        ),
        scratch_types=[
            pltpu.VMEM((S_LEN, LANES), jnp.float32),
            pltpu.VMEM((N_GROUPS, LANES), jnp.float32),
            pltpu.VMEM((LIST_CAP, LANES), jnp.int32),
            pltpu.VMEM((K_OUT, LANES), jnp.float32),
            [pltpu.SemaphoreType.DMA] * N_CHUNKS,
        ],
    )(inputs)


def kernel(inputs):
    b, s, c = inputs.shape
    out = _sc_call(inputs)
    out = out.reshape(b, c // LANES, K_OUT, LANES)
    return jnp.transpose(out, (0, 1, 3, 2)).reshape(b, c, K_OUT)


# speedup vs baseline: 1.0606x; 1.0606x over previous
import functools

import jax
import jax.numpy as jnp
from jax import lax
from jax.experimental import pallas as pl
from jax.experimental.pallas import tpu as pltpu
from jax.experimental.pallas import tpu_sc as plsc

K_OUT = 8
LIST_CAP = 10
LANES = 16
S_LEN = 4096
N_GROUPS = S_LEN // LANES
N_CHUNKS = 8
ROWS_PER_CHUNK = S_LEN // N_CHUNKS
GROUPS_PER_CHUNK = N_GROUPS // N_CHUNKS


def _insert(ms, x):
    new = []
    for m in ms:
        hi = jnp.maximum(m, x)
        x = jnp.minimum(m, x)
        new.append(hi)
    return new


def _sc_body(in_hbm, out_hbm, xbuf, gmbuf, glist, obuf, sem):
    info = plsc.get_sparse_core_info()
    nc, ns = info.num_cores, info.num_subcores
    wid = lax.axis_index("s") * nc + lax.axis_index("c")
    tasks_per_tile = (4 * 64) // (nc * ns)
    lane_iota = lax.broadcasted_iota(jnp.int32, (LANES,), 0)
    neg_inf = jnp.full((LANES,), -jnp.inf, dtype=jnp.float32)

    def task_ids(t):
        task = wid * tasks_per_tile + t
        return task, task // 64, task % 64

    def start_task_dmas(t):
        _, b, cg = task_ids(t)
        handles = []
        for q in range(N_CHUNKS):
            handles.append(
                pltpu.async_copy(
                    in_hbm.at[
                        b,
                        pl.ds(q * ROWS_PER_CHUNK, ROWS_PER_CHUNK),
                        pl.ds(cg * LANES, LANES),
                    ],
                    xbuf.at[pl.ds(q * ROWS_PER_CHUNK, ROWS_PER_CHUNK), :],
                    sem[q],
                )
            )
        return handles

    zero16 = jnp.zeros((LANES,), jnp.int32)
    for r in range(LIST_CAP):
        glist[r] = zero16

    handles = start_task_dmas(0)

    for t in range(tasks_per_tile):
        task, _, _ = task_ids(t)

        ms = (neg_inf,) * K_OUT
        for q in range(N_CHUNKS):
            handles[q].wait()

            def group_step(g, ms):
                base = g * LANES
                m01 = jnp.maximum(xbuf[base + 0], xbuf[base + 1])
                m23 = jnp.maximum(xbuf[base + 2], xbuf[base + 3])
                m45 = jnp.maximum(xbuf[base + 4], xbuf[base + 5])
                m67 = jnp.maximum(xbuf[base + 6], xbuf[base + 7])
                m89 = jnp.maximum(xbuf[base + 8], xbuf[base + 9])
                mab = jnp.maximum(xbuf[base + 10], xbuf[base + 11])
                mcd = jnp.maximum(xbuf[base + 12], xbuf[base + 13])
                mef = jnp.maximum(xbuf[base + 14], xbuf[base + 15])
                m = jnp.maximum(
                    jnp.maximum(jnp.maximum(m01, m23), jnp.maximum(m45, m67)),
                    jnp.maximum(jnp.maximum(m89, mab), jnp.maximum(mcd, mef)),
                )
                gmbuf[g] = m
                return tuple(_insert(list(ms), m))

            ms = lax.fori_loop(
                q * GROUPS_PER_CHUNK,
                (q + 1) * GROUPS_PER_CHUNK,
                group_step,
                ms,
                unroll=2,
            )
        t8 = ms[K_OUT - 1]

        def list_step(g, cnt):
            sel = jnp.logical_and(gmbuf[g] >= t8, cnt < LIST_CAP)
            gvec = jnp.full((LANES,), g, dtype=jnp.int32)
            plsc.store_scatter(glist, [cnt, lane_iota], gvec, mask=sel)
            return cnt + jnp.where(sel, 1, 0)

        cntf = lax.fori_loop(
            0, N_GROUPS, list_step, jnp.zeros((LANES,), jnp.int32), unroll=2
        )

        def extract_step(j, carry):
            msa, msb = carry
            rows = glist[j] * LANES
            valid = j < cntf
            msa, msb = list(msa), list(msb)
            for v in range(0, LANES, 2):
                xa = plsc.load_gather(xbuf, [rows + v, lane_iota])
                xb = plsc.load_gather(xbuf, [rows + v + 1, lane_iota])
                msa = _insert(msa, jnp.where(valid, xa, neg_inf))
                msb = _insert(msb, jnp.where(valid, xb, neg_inf))
            return (tuple(msa), tuple(msb))

        msa, msb = lax.fori_loop(
            0, LIST_CAP, extract_step, ((neg_inf,) * K_OUT,) * 2
        )
        ms2 = list(msa)
        for x in msb:
            ms2 = _insert(ms2, x)

        for k in range(K_OUT):
            obuf[k] = ms2[k]
        pltpu.sync_copy(obuf, out_hbm.at[task])

        if t + 1 < tasks_per_tile:
            handles = start_task_dmas(t + 1)


def _sc_call(inputs):
    mesh = plsc.VectorSubcoreMesh(core_axis_name="c", subcore_axis_name="s")
    return pl.kernel(
        _sc_body,
        out_type=jax.ShapeDtypeStruct((4 * 64, K_OUT, LANES), jnp.float32),
        mesh=mesh,
        compiler_params=pltpu.CompilerParams(
            use_tc_tiling_on_sc=False, needs_layout_passes=False
        ),
        scratch_types=[
            pltpu.VMEM((S_LEN, LANES), jnp.float32),
            pltpu.VMEM((N_GROUPS, LANES), jnp.float32),
            pltpu.VMEM((LIST_CAP, LANES), jnp.int32),
            pltpu.VMEM((K_OUT, LANES), jnp.float32),
            [pltpu.SemaphoreType.DMA] * N_CHUNKS,
        ],
    )(inputs)


def kernel(inputs):
    b, s, c = inputs.shape
    out = _sc_call(inputs)
    out = out.reshape(b, c // LANES, K_OUT, LANES)
    return jnp.transpose(out, (0, 1, 3, 2)).reshape(b, c, K_OUT)
